# baseline (speedup 1.0000x reference)
import functools

import jax
import jax.numpy as jnp
from jax import lax
from jax.experimental import pallas as pl
from jax.experimental.pallas import tpu as pltpu
from jax.experimental.pallas import tpu_sc as plsc

N = 10000
D = 128
E = 320000
NC = 2
NS = 16
NW = NC * NS
B = 128
NBUF = 2
TILE_ROWS = 632
N_PAD = NS * TILE_ROWS
CHUNKS = 160
SEG = 4
SEGC = 16
EDGES_PAD = NW * CHUNKS * B
BN = 1000
GRID_N = N // BN
CNT_WORDS = 16384
CNT_TILE = CNT_WORDS // NS


def _prop_body(h_ref, w_ref, out_ref):
    out_ref[...] = lax.dot_general(
        h_ref[...], w_ref[...], (((1,), (1,)), ((), ())),
        preferred_element_type=jnp.float32)


_prop_call = pl.pallas_call(
    _prop_body,
    grid=(2, GRID_N),
    in_specs=[
        pl.BlockSpec((BN, D), lambda t, i: (i, 0)),
        pl.BlockSpec((D, D), lambda t, i: (t, 0)),
    ],
    out_specs=pl.BlockSpec((BN, D), lambda t, i: (t * GRID_N + i, 0)),
    out_shape=jax.ShapeDtypeStruct((2 * N, D), jnp.float32),
)


def _sc_mesh():
    return plsc.VectorSubcoreMesh(
        core_axis_name="c", subcore_axis_name="s",
        num_cores=NC, num_subcores=NS)


@functools.cache
def _make_sc_msg():
    return functools.partial(
        pl.kernel,
        out_type=(
            jax.ShapeDtypeStruct((NC, N_PAD, D), jnp.float32),
            jax.ShapeDtypeStruct((NC * CNT_WORDS,), jnp.float32),
        ),
        mesh=_sc_mesh(),
        scratch_types=[
            pltpu.VMEM((SEG, B), jnp.int32),
            pltpu.VMEM((SEG, B), jnp.int32),
            pltpu.VMEM((SEG, B), jnp.int32),
            pltpu.VMEM((SEG, B), jnp.int32),
            pltpu.VMEM((NBUF, B, D), jnp.float32),
            pltpu.VMEM((B,), jnp.float32),
            pltpu.VMEM((CNT_TILE,), jnp.float32),
            pltpu.VMEM_SHARED((N_PAD, D), jnp.float32),
            pltpu.VMEM_SHARED((CNT_WORDS,), jnp.float32),
            pltpu.SemaphoreType.DMA,
            pltpu.SemaphoreType.DMA,
            pltpu.SemaphoreType.DMA,
            pltpu.SemaphoreType.DMA,
            pltpu.SemaphoreType.DMA,
            pltpu.SemaphoreType.DMA,
        ],
    )(_sc_msg_body)


PAIRS = CHUNKS // (2 * SEG)


def _sc_msg_body(table, gidx, sidx, msg_out, cnt_out,
                 gv0, sv0, gv1, sv1, rows, ones_v, zbuf, accm, accc,
                 gsem0, gsem1, ssem0, ssem1, isem0, isem1):
    c = lax.axis_index("c")
    s = lax.axis_index("s")
    w = s * NC + c
    base = s * TILE_ROWS
    cbase = s * CNT_TILE
    gsem = (gsem0, gsem1)
    ssem = (ssem0, ssem1)
    ig0 = pltpu.async_copy(gidx.at[w, pl.ds(0, SEG)], gv0, isem0)
    is0 = pltpu.async_copy(sidx.at[w, pl.ds(0, SEG)], sv0, isem0)
    ig1 = pltpu.async_copy(gidx.at[w, pl.ds(SEG, SEG)], gv1, isem1)
    is1 = pltpu.async_copy(sidx.at[w, pl.ds(SEG, SEG)], sv1, isem1)
    one = jnp.full((16,), 1.0, jnp.float32)
    zero = jnp.zeros((16,), jnp.float32)
    for k in range(B // 16):
        ones_v[pl.ds(k * 16, 16)] = one
    for k in range(CNT_TILE // 16):
        zbuf[pl.ds(k * 16, 16)] = zero

    def zrow(r, carry):
        for k in range(D // 16):
            rows[0, r, pl.ds(k * 16, 16)] = zero
        return carry

    lax.fori_loop(0, B, zrow, 0)
    for blk in range((TILE_ROWS + B - 1) // B):
        nrows = min(B, TILE_ROWS - blk * B)
        pltpu.sync_copy(rows.at[0, pl.ds(0, nrows)],
                        accm.at[pl.ds(base + blk * B, nrows)])
    pltpu.sync_copy(zbuf, accc.at[pl.ds(cbase, CNT_TILE)])
    plsc.subcore_barrier()
    ig0.wait()
    is0.wait()
    ig1.wait()
    is1.wait()
    pltpu.async_copy(table.at[gv0.at[0]], rows.at[0], gsem0)
    pltpu.async_copy(table.at[gv0.at[1]], rows.at[1], gsem1)

    def pair(t, carry):
        for k in range(2 * SEG):
            b = k % 2
            gvk, svk = (gv0, sv0) if k < SEG else (gv1, sv1)
            r = k % SEG
            pltpu.make_async_copy(table.at[gvk.at[r]], rows.at[b],
                                  gsem[b]).wait()
            sc = pltpu.async_copy(rows.at[b], accm.at[svk.at[r]], ssem[b],
                                  add=True)
            cd = pltpu.async_copy(ones_v, accc.at[svk.at[r]], ssem[b],
                                  add=True)
            sc.wait()
            cd.wait()
            if k == SEG - 1:
                @pl.when(t < PAIRS - 1)
                def _():
                    pltpu.async_copy(
                        gidx.at[w, pl.ds((2 * t + 2) * SEG, SEG)], gv0,
                        isem0)
                    pltpu.async_copy(
                        sidx.at[w, pl.ds((2 * t + 2) * SEG, SEG)], sv0,
                        isem0)
            if k == 2 * SEG - 1:
                @pl.when(t < PAIRS - 1)
                def _():
                    pltpu.async_copy(
                        gidx.at[w, pl.ds((2 * t + 3) * SEG, SEG)], gv1,
                        isem1)
                    pltpu.async_copy(
                        sidx.at[w, pl.ds((2 * t + 3) * SEG, SEG)], sv1,
                        isem1)
            nk = k + 2
            if nk < SEG:
                pltpu.async_copy(table.at[gv0.at[nk]], rows.at[b], gsem[b])
            elif nk == SEG:
                @pl.when(t > 0)
                def _():
                    pltpu.make_async_copy(
                        gidx.at[w, pl.ds((2 * t + 1) * SEG, SEG)], gv1,
                        isem1).wait()
                    pltpu.make_async_copy(
                        sidx.at[w, pl.ds((2 * t + 1) * SEG, SEG)], sv1,
                        isem1).wait()
                pltpu.async_copy(table.at[gv1.at[0]], rows.at[b], gsem[b])
            elif nk < 2 * SEG:
                pltpu.async_copy(table.at[gv1.at[nk - SEG]], rows.at[b],
                                 gsem[b])
            else:
                @pl.when(t < PAIRS - 1)
                def _():
                    if nk == 2 * SEG:
                        pltpu.make_async_copy(
                            gidx.at[w, pl.ds((2 * t + 2) * SEG, SEG)], gv0,
                            isem0).wait()
                        pltpu.make_async_copy(
                            sidx.at[w, pl.ds((2 * t + 2) * SEG, SEG)], sv0,
                            isem0).wait()
                    pltpu.async_copy(table.at[gv0.at[nk - 2 * SEG]],
                                     rows.at[b], gsem[b])
        return carry

    lax.fori_loop(0, PAIRS, pair, 0)
    plsc.subcore_barrier()
    pltpu.sync_copy(accm.at[pl.ds(base, TILE_ROWS)],
                    msg_out.at[c, pl.ds(base, TILE_ROWS)])
    pltpu.sync_copy(accc.at[pl.ds(cbase, CNT_TILE)],
                    cnt_out.at[pl.ds(c * CNT_WORDS + cbase, CNT_TILE)])


def _gru_body(p_ref, c0_ref, c1_ref, h_ref, wih_ref, whh_ref, bih_ref,
              bhh_ref, out_ref):
    cnt = c0_ref[...] + c1_ref[...]
    inv = jnp.where(cnt == 0.0, 1.0, 1.0 / cnt)
    m = (p_ref[0] + p_ref[1]) * inv
    h = h_ref[...]
    gi = lax.dot_general(m, wih_ref[...], (((1,), (1,)), ((), ())),
                         preferred_element_type=jnp.float32) + bih_ref[...]
    gh = lax.dot_general(h, whh_ref[...], (((1,), (1,)), ((), ())),
                         preferred_element_type=jnp.float32) + bhh_ref[...]
    i_r, i_z, i_n = gi[:, :D], gi[:, D:2 * D], gi[:, 2 * D:]
    h_r, h_z, h_n = gh[:, :D], gh[:, D:2 * D], gh[:, 2 * D:]
    r = jax.nn.sigmoid(i_r + h_r)
    z = jax.nn.sigmoid(i_z + h_z)
    nn = jnp.tanh(i_n + r * h_n)
    out_ref[...] = (1.0 - z) * nn + z * h


_gru_call = pl.pallas_call(
    _gru_body,
    grid=(GRID_N,),
    in_specs=[
        pl.BlockSpec((NC, BN, D), lambda i: (0, i, 0)),
        pl.BlockSpec((BN, 1), lambda i: (i, 0)),
        pl.BlockSpec((BN, 1), lambda i: (i, 0)),
        pl.BlockSpec((BN, D), lambda i: (i, 0)),
        pl.BlockSpec((3 * D, D), lambda i: (0, 0)),
        pl.BlockSpec((3 * D, D), lambda i: (0, 0)),
        pl.BlockSpec((1, 3 * D), lambda i: (0, 0)),
        pl.BlockSpec((1, 3 * D), lambda i: (0, 0)),
    ],
    out_specs=pl.BlockSpec((BN, D), lambda i: (i, 0)),
    out_shape=jax.ShapeDtypeStruct((N, D), jnp.float32),
)


def _gru_prop_body(p_ref, c0_ref, c1_ref, h_ref, wih_ref, whh_ref, bih_ref,
                   bhh_ref, wmsg_ref, outh_ref, outt_ref):
    cnt = c0_ref[...] + c1_ref[...]
    inv = jnp.where(cnt == 0.0, 1.0, 1.0 / cnt)
    m = (p_ref[0] + p_ref[1]) * inv
    h = h_ref[...]
    gi = lax.dot_general(m, wih_ref[...], (((1,), (1,)), ((), ())),
                         preferred_element_type=jnp.float32) + bih_ref[...]
    gh = lax.dot_general(h, whh_ref[...], (((1,), (1,)), ((), ())),
                         preferred_element_type=jnp.float32) + bhh_ref[...]
    i_r, i_z, i_n = gi[:, :D], gi[:, D:2 * D], gi[:, 2 * D:]
    h_r, h_z, h_n = gh[:, :D], gh[:, D:2 * D], gh[:, 2 * D:]
    r = jax.nn.sigmoid(i_r + h_r)
    z = jax.nn.sigmoid(i_z + h_z)
    nn = jnp.tanh(i_n + r * h_n)
    hn = (1.0 - z) * nn + z * h
    outh_ref[...] = hn
    outt_ref[0] = lax.dot_general(
        hn, wmsg_ref[pl.ds(0, D), :], (((1,), (1,)), ((), ())),
        preferred_element_type=jnp.float32)
    outt_ref[1] = lax.dot_general(
        hn, wmsg_ref[pl.ds(D, D), :], (((1,), (1,)), ((), ())),
        preferred_element_type=jnp.float32)


_gru_prop_call = pl.pallas_call(
    _gru_prop_body,
    grid=(GRID_N,),
    in_specs=[
        pl.BlockSpec((NC, BN, D), lambda i: (0, i, 0)),
        pl.BlockSpec((BN, 1), lambda i: (i, 0)),
        pl.BlockSpec((BN, 1), lambda i: (i, 0)),
        pl.BlockSpec((BN, D), lambda i: (i, 0)),
        pl.BlockSpec((3 * D, D), lambda i: (0, 0)),
        pl.BlockSpec((3 * D, D), lambda i: (0, 0)),
        pl.BlockSpec((1, 3 * D), lambda i: (0, 0)),
        pl.BlockSpec((1, 3 * D), lambda i: (0, 0)),
        pl.BlockSpec((2 * D, D), lambda i: (0, 0)),
    ],
    out_specs=[
        pl.BlockSpec((BN, D), lambda i: (i, 0)),
        pl.BlockSpec((2, BN, D), lambda i: (0, i, 0)),
    ],
    out_shape=[
        jax.ShapeDtypeStruct((N, D), jnp.float32),
        jax.ShapeDtypeStruct((2, N, D), jnp.float32),
    ],
)


def kernel(node_states, edge_list, W_msg, W_ih, W_hh, b_ih, b_hh):
    src = edge_list[:, 0]
    dst = edge_list[:, 1]
    pad = EDGES_PAD - 2 * E
    ar = jnp.arange(pad, dtype=jnp.int32)
    pad_g = ar % (2 * N)
    pad_s = N + (ar % (N_PAD - N))
    gidx = jnp.concatenate([src, dst + N, pad_g]).reshape(NW, CHUNKS, B)
    sidx = jnp.concatenate([dst, src, pad_s]).reshape(NW, CHUNKS, B)
    bih = b_ih.reshape(1, 3 * D)
    bhh = b_hh.reshape(1, 3 * D)

    h = node_states
    table = _prop_call(h, W_msg)
    msg, cntflat = _make_sc_msg()(table, gidx, sidx)
    c0 = cntflat[0:N].reshape(N, 1)
    c1 = cntflat[CNT_WORDS:CNT_WORDS + N].reshape(N, 1)
    h, table3 = _gru_prop_call(msg, c0, c1, h, W_ih, W_hh, bih, bhh, W_msg)
    msg, _ = _make_sc_msg()(table3.reshape(2 * N, D), gidx, sidx)
    h = _gru_call(msg, c0, c1, h, W_ih, W_hh, bih, bhh)
    return h


# speedup vs baseline: 1.0222x; 1.0222x over previous
import functools

import jax
import jax.numpy as jnp
from jax import lax
from jax.experimental import pallas as pl
from jax.experimental.pallas import tpu as pltpu
from jax.experimental.pallas import tpu_sc as plsc

N = 10000
D = 128
E = 320000
NC = 2
NS = 16
NW = NC * NS
B = 128
NBUF = 2
TILE_ROWS = 632
N_PAD = NS * TILE_ROWS
CHUNKS = 160
SEG = 4
SEGC = 16
EDGES_PAD = NW * CHUNKS * B
BN = 1000
GRID_N = N // BN


def _prop_body(h_ref, w_ref, out_ref):
    out_ref[...] = lax.dot_general(
        h_ref[...], w_ref[...], (((1,), (1,)), ((), ())),
        preferred_element_type=jnp.float32)


_prop_call = pl.pallas_call(
    _prop_body,
    grid=(2, GRID_N),
    in_specs=[
        pl.BlockSpec((BN, D), lambda t, i: (i, 0)),
        pl.BlockSpec((D, D), lambda t, i: (t, 0)),
    ],
    out_specs=pl.BlockSpec((BN, D), lambda t, i: (t * GRID_N + i, 0)),
    out_shape=jax.ShapeDtypeStruct((2 * N, D), jnp.float32),
)


def _sc_mesh():
    return plsc.VectorSubcoreMesh(
        core_axis_name="c", subcore_axis_name="s",
        num_cores=NC, num_subcores=NS)


@functools.cache
def _make_sc_msg():
    return functools.partial(
        pl.kernel,
        out_type=jax.ShapeDtypeStruct((NC, N_PAD, D), jnp.float32),
        mesh=_sc_mesh(),
        scratch_types=[
            pltpu.VMEM((SEG, B), jnp.int32),
            pltpu.VMEM((SEG, B), jnp.int32),
            pltpu.VMEM((SEG, B), jnp.int32),
            pltpu.VMEM((SEG, B), jnp.int32),
            pltpu.VMEM((NBUF, B, D), jnp.float32),
            pltpu.VMEM_SHARED((N_PAD, D), jnp.float32),
            pltpu.SemaphoreType.DMA,
            pltpu.SemaphoreType.DMA,
            pltpu.SemaphoreType.DMA,
            pltpu.SemaphoreType.DMA,
            pltpu.SemaphoreType.DMA,
            pltpu.SemaphoreType.DMA,
        ],
    )(_sc_msg_body)


PAIRS = CHUNKS // (2 * SEG)


def _sc_msg_body(table, gidx, sidx, zmsg, msg_out,
                 gv0, sv0, gv1, sv1, rows, accm,
                 gsem0, gsem1, ssem0, ssem1, isem0, isem1):
    c = lax.axis_index("c")
    s = lax.axis_index("s")
    w = s * NC + c
    base = s * TILE_ROWS
    gsem = (gsem0, gsem1)
    ssem = (ssem0, ssem1)
    ig0 = pltpu.async_copy(gidx.at[w, pl.ds(0, SEG)], gv0, isem0)
    is0 = pltpu.async_copy(sidx.at[w, pl.ds(0, SEG)], sv0, isem0)
    ig1 = pltpu.async_copy(gidx.at[w, pl.ds(SEG, SEG)], gv1, isem1)
    is1 = pltpu.async_copy(sidx.at[w, pl.ds(SEG, SEG)], sv1, isem1)
    pltpu.sync_copy(zmsg.at[pl.ds(base, TILE_ROWS)],
                    accm.at[pl.ds(base, TILE_ROWS)])
    plsc.subcore_barrier()
    ig0.wait()
    is0.wait()
    ig1.wait()
    is1.wait()
    pltpu.async_copy(table.at[gv0.at[0]], rows.at[0], gsem0)
    pltpu.async_copy(table.at[gv0.at[1]], rows.at[1], gsem1)

    def pair(t, carry):
        for k in range(2 * SEG):
            b = k % 2
            gvk, svk = (gv0, sv0) if k < SEG else (gv1, sv1)
            r = k % SEG
            pltpu.make_async_copy(table.at[gvk.at[r]], rows.at[b],
                                  gsem[b]).wait()
            sc = pltpu.async_copy(rows.at[b], accm.at[svk.at[r]], ssem[b],
                                  add=True)
            sc.wait()
            if k == SEG - 1:
                @pl.when(t < PAIRS - 1)
                def _():
                    pltpu.async_copy(
                        gidx.at[w, pl.ds((2 * t + 2) * SEG, SEG)], gv0,
                        isem0)
                    pltpu.async_copy(
                        sidx.at[w, pl.ds((2 * t + 2) * SEG, SEG)], sv0,
                        isem0)
            if k == 2 * SEG - 1:
                @pl.when(t < PAIRS - 1)
                def _():
                    pltpu.async_copy(
                        gidx.at[w, pl.ds((2 * t + 3) * SEG, SEG)], gv1,
                        isem1)
                    pltpu.async_copy(
                        sidx.at[w, pl.ds((2 * t + 3) * SEG, SEG)], sv1,
                        isem1)
            nk = k + 2
            if nk < SEG:
                pltpu.async_copy(table.at[gv0.at[nk]], rows.at[b], gsem[b])
            elif nk == SEG:
                @pl.when(t > 0)
                def _():
                    pltpu.make_async_copy(
                        gidx.at[w, pl.ds((2 * t + 1) * SEG, SEG)], gv1,
                        isem1).wait()
                    pltpu.make_async_copy(
                        sidx.at[w, pl.ds((2 * t + 1) * SEG, SEG)], sv1,
                        isem1).wait()
                pltpu.async_copy(table.at[gv1.at[0]], rows.at[b], gsem[b])
            elif nk < 2 * SEG:
                pltpu.async_copy(table.at[gv1.at[nk - SEG]], rows.at[b],
                                 gsem[b])
            else:
                @pl.when(t < PAIRS - 1)
                def _():
                    if nk == 2 * SEG:
                        pltpu.make_async_copy(
                            gidx.at[w, pl.ds((2 * t + 2) * SEG, SEG)], gv0,
                            isem0).wait()
                        pltpu.make_async_copy(
                            sidx.at[w, pl.ds((2 * t + 2) * SEG, SEG)], sv0,
                            isem0).wait()
                    pltpu.async_copy(table.at[gv0.at[nk - 2 * SEG]],
                                     rows.at[b], gsem[b])
        return carry

    lax.fori_loop(0, PAIRS, pair, 0)
    plsc.subcore_barrier()
    pltpu.sync_copy(accm.at[pl.ds(base, TILE_ROWS)],
                    msg_out.at[c, pl.ds(base, TILE_ROWS)])


CNT_WORDS = 16384
CNT_TILE = CNT_WORDS // NS


@functools.cache
def _make_sc_cnt():
    return functools.partial(
        pl.kernel,
        out_type=jax.ShapeDtypeStruct((NC * CNT_WORDS,), jnp.float32),
        mesh=_sc_mesh(),
        scratch_types=[
            pltpu.VMEM((SEGC, B), jnp.int32),
            pltpu.VMEM((B,), jnp.float32),
            pltpu.VMEM_SHARED((CNT_WORDS,), jnp.float32),
            pltpu.SemaphoreType.DMA,
        ],
    )(_sc_cnt_body)


def _sc_cnt_body(sidx, zcnt, cnt_out, sv, ones_v, accc, sem):
    c = lax.axis_index("c")
    s = lax.axis_index("s")
    w = s * NC + c
    base = s * CNT_TILE
    one = jnp.full((16,), 1.0, jnp.float32)
    for k in range(B // 16):
        ones_v[pl.ds(k * 16, 16)] = one
    pltpu.sync_copy(zcnt.at[pl.ds(base, CNT_TILE)],
                    accc.at[pl.ds(base, CNT_TILE)])
    plsc.subcore_barrier()

    def stage(t, carry):
        pltpu.sync_copy(sidx.at[w, pl.ds(t * SEGC, SEGC)], sv)
        descs = []
        for j in range(SEGC):
            descs.append(
                pltpu.async_copy(ones_v, accc.at[sv.at[j]], sem, add=True))
        for dsc in descs:
            dsc.wait()
        return carry

    lax.fori_loop(0, CHUNKS // SEGC, stage, 0)
    plsc.subcore_barrier()
    pltpu.sync_copy(accc.at[pl.ds(base, CNT_TILE)],
                    cnt_out.at[pl.ds(c * CNT_WORDS + base, CNT_TILE)])


def _gru_body(p_ref, c0_ref, c1_ref, h_ref, wih_ref, whh_ref, bih_ref,
              bhh_ref, out_ref):
    cnt = c0_ref[...] + c1_ref[...]
    inv = jnp.where(cnt == 0.0, 1.0, 1.0 / cnt)
    m = (p_ref[0] + p_ref[1]) * inv
    h = h_ref[...]
    gi = lax.dot_general(m, wih_ref[...], (((1,), (1,)), ((), ())),
                         preferred_element_type=jnp.float32) + bih_ref[...]
    gh = lax.dot_general(h, whh_ref[...], (((1,), (1,)), ((), ())),
                         preferred_element_type=jnp.float32) + bhh_ref[...]
    i_r, i_z, i_n = gi[:, :D], gi[:, D:2 * D], gi[:, 2 * D:]
    h_r, h_z, h_n = gh[:, :D], gh[:, D:2 * D], gh[:, 2 * D:]
    r = jax.nn.sigmoid(i_r + h_r)
    z = jax.nn.sigmoid(i_z + h_z)
    nn = jnp.tanh(i_n + r * h_n)
    out_ref[...] = (1.0 - z) * nn + z * h


_gru_call = pl.pallas_call(
    _gru_body,
    grid=(GRID_N,),
    in_specs=[
        pl.BlockSpec((NC, BN, D), lambda i: (0, i, 0)),
        pl.BlockSpec((BN, 1), lambda i: (i, 0)),
        pl.BlockSpec((BN, 1), lambda i: (i, 0)),
        pl.BlockSpec((BN, D), lambda i: (i, 0)),
        pl.BlockSpec((3 * D, D), lambda i: (0, 0)),
        pl.BlockSpec((3 * D, D), lambda i: (0, 0)),
        pl.BlockSpec((1, 3 * D), lambda i: (0, 0)),
        pl.BlockSpec((1, 3 * D), lambda i: (0, 0)),
    ],
    out_specs=pl.BlockSpec((BN, D), lambda i: (i, 0)),
    out_shape=jax.ShapeDtypeStruct((N, D), jnp.float32),
)


def _gru_prop_body(p_ref, c0_ref, c1_ref, h_ref, wih_ref, whh_ref, bih_ref,
                   bhh_ref, wmsg_ref, outh_ref, outt_ref):
    cnt = c0_ref[...] + c1_ref[...]
    inv = jnp.where(cnt == 0.0, 1.0, 1.0 / cnt)
    m = (p_ref[0] + p_ref[1]) * inv
    h = h_ref[...]
    gi = lax.dot_general(m, wih_ref[...], (((1,), (1,)), ((), ())),
                         preferred_element_type=jnp.float32) + bih_ref[...]
    gh = lax.dot_general(h, whh_ref[...], (((1,), (1,)), ((), ())),
                         preferred_element_type=jnp.float32) + bhh_ref[...]
    i_r, i_z, i_n = gi[:, :D], gi[:, D:2 * D], gi[:, 2 * D:]
    h_r, h_z, h_n = gh[:, :D], gh[:, D:2 * D], gh[:, 2 * D:]
    r = jax.nn.sigmoid(i_r + h_r)
    z = jax.nn.sigmoid(i_z + h_z)
    nn = jnp.tanh(i_n + r * h_n)
    hn = (1.0 - z) * nn + z * h
    outh_ref[...] = hn
    outt_ref[0] = lax.dot_general(
        hn, wmsg_ref[pl.ds(0, D), :], (((1,), (1,)), ((), ())),
        preferred_element_type=jnp.float32)
    outt_ref[1] = lax.dot_general(
        hn, wmsg_ref[pl.ds(D, D), :], (((1,), (1,)), ((), ())),
        preferred_element_type=jnp.float32)


_gru_prop_call = pl.pallas_call(
    _gru_prop_body,
    grid=(GRID_N,),
    in_specs=[
        pl.BlockSpec((NC, BN, D), lambda i: (0, i, 0)),
        pl.BlockSpec((BN, 1), lambda i: (i, 0)),
        pl.BlockSpec((BN, 1), lambda i: (i, 0)),
        pl.BlockSpec((BN, D), lambda i: (i, 0)),
        pl.BlockSpec((3 * D, D), lambda i: (0, 0)),
        pl.BlockSpec((3 * D, D), lambda i: (0, 0)),
        pl.BlockSpec((1, 3 * D), lambda i: (0, 0)),
        pl.BlockSpec((1, 3 * D), lambda i: (0, 0)),
        pl.BlockSpec((2 * D, D), lambda i: (0, 0)),
    ],
    out_specs=[
        pl.BlockSpec((BN, D), lambda i: (i, 0)),
        pl.BlockSpec((2, BN, D), lambda i: (0, i, 0)),
    ],
    out_shape=[
        jax.ShapeDtypeStruct((N, D), jnp.float32),
        jax.ShapeDtypeStruct((2, N, D), jnp.float32),
    ],
)


def kernel(node_states, edge_list, W_msg, W_ih, W_hh, b_ih, b_hh):
    src = edge_list[:, 0]
    dst = edge_list[:, 1]
    pad = EDGES_PAD - 2 * E
    ar = jnp.arange(pad, dtype=jnp.int32)
    pad_g = ar % (2 * N)
    pad_s = N + (ar % (N_PAD - N))
    gidx = jnp.concatenate([src, dst + N, pad_g]).reshape(NW, CHUNKS, B)
    sidx = jnp.concatenate([dst, src, pad_s]).reshape(NW, CHUNKS, B)
    zmsg = jnp.zeros((N_PAD, D), jnp.float32)
    zcnt = jnp.zeros((NC * CNT_WORDS,), jnp.float32)
    bih = b_ih.reshape(1, 3 * D)
    bhh = b_hh.reshape(1, 3 * D)

    cntflat = _make_sc_cnt()(sidx, zcnt)
    c0 = cntflat[0:N].reshape(N, 1)
    c1 = cntflat[CNT_WORDS:CNT_WORDS + N].reshape(N, 1)
    h = node_states
    table = _prop_call(h, W_msg)
    msg = _make_sc_msg()(table, gidx, sidx, zmsg)
    h, table3 = _gru_prop_call(msg, c0, c1, h, W_ih, W_hh, bih, bhh, W_msg)
    msg = _make_sc_msg()(table3.reshape(2 * N, D), gidx, sidx, zmsg)
    h = _gru_call(msg, c0, c1, h, W_ih, W_hh, bih, bhh)
    return h
